# Initial kernel scaffold; baseline (speedup 1.0000x reference)
import functools

import jax
import jax.numpy as jnp
from jax import lax
from jax.experimental import pallas as pl
from jax.experimental.pallas import tpu as pltpu
from jax.experimental.pallas import tpu_sc as plsc

N = 100000
D_IN = 144
D_OUT = 128
HID = 64
NUM_SEG = 1024

RBLK = 2000
NBLK = N // RBLK

CHUNK = 128
NCH_FULL = N // CHUNK
TAIL = N - NCH_FULL * CHUNK
NW = 32
KMAX = (NCH_FULL + NW - 1) // NW
SEG_PER_TILE = NUM_SEG // 16


def _max_body(z_ref, w1_ref, b1_ref, w2_ref, m_ref, m_acc):
    i = pl.program_id(0)
    h = jnp.maximum(
        jnp.dot(z_ref[...], w1_ref[...], preferred_element_type=jnp.float32)
        + b1_ref[...],
        0.0,
    )
    s = jnp.dot(h, w2_ref[...], preferred_element_type=jnp.float32)
    bm = jnp.max(s)

    @pl.when(i == 0)
    def _():
        m_acc[0] = bm

    @pl.when(i > 0)
    def _():
        m_acc[0] = jnp.maximum(m_acc[0], bm)

    @pl.when(i == NBLK - 1)
    def _():
        m_ref[0, 0] = m_acc[0]


def _scores_max(z, W1, b1, W2):
    return pl.pallas_call(
        _max_body,
        grid=(NBLK,),
        in_specs=[
            pl.BlockSpec((RBLK, D_IN), lambda i: (i, 0)),
            pl.BlockSpec((D_IN, HID), lambda i: (0, 0)),
            pl.BlockSpec((1, HID), lambda i: (0, 0)),
            pl.BlockSpec((HID, 1), lambda i: (0, 0)),
        ],
        out_specs=pl.BlockSpec((1, 1), lambda i: (0, 0)),
        out_shape=jax.ShapeDtypeStruct((1, 1), jnp.float32),
        scratch_shapes=[pltpu.SMEM((1,), jnp.float32)],
    )(z, W1, b1, W2)


def _vals_body(z_ref, zl_ref, w1_ref, b1_ref, w2_ref, b2_ref, m_ref,
               vals_ref, zsum_ref, z_acc):
    i = pl.program_id(0)
    h = jnp.maximum(
        jnp.dot(z_ref[...], w1_ref[...], preferred_element_type=jnp.float32)
        + b1_ref[...],
        0.0,
    )
    s = jnp.dot(h, w2_ref[...], preferred_element_type=jnp.float32) + b2_ref[0]
    e = jnp.exp(s - m_ref[0])
    vals_ref[...] = e * zl_ref[...]
    bs = jnp.sum(e)

    @pl.when(i == 0)
    def _():
        z_acc[0] = bs

    @pl.when(i > 0)
    def _():
        z_acc[0] = z_acc[0] + bs

    @pl.when(i == NBLK - 1)
    def _():
        zsum_ref[0, 0] = z_acc[0]


def _weighted_vals(z, W1, b1, W2, b2, m):
    return pl.pallas_call(
        _vals_body,
        grid=(NBLK,),
        in_specs=[
            pl.BlockSpec((RBLK, D_IN), lambda i: (i, 0)),
            pl.BlockSpec((RBLK, D_OUT), lambda i: (i, 0)),
            pl.BlockSpec((D_IN, HID), lambda i: (0, 0)),
            pl.BlockSpec((1, HID), lambda i: (0, 0)),
            pl.BlockSpec((HID, 1), lambda i: (0, 0)),
            pl.BlockSpec(memory_space=pltpu.SMEM),
            pl.BlockSpec(memory_space=pltpu.SMEM),
        ],
        out_specs=[
            pl.BlockSpec((RBLK, D_OUT), lambda i: (i, 0)),
            pl.BlockSpec((1, 1), lambda i: (0, 0)),
        ],
        out_shape=[
            jax.ShapeDtypeStruct((N, D_OUT), jnp.float32),
            jax.ShapeDtypeStruct((1, 1), jnp.float32),
        ],
        scratch_shapes=[pltpu.SMEM((1,), jnp.float32)],
    )(z, z, W1, b1, W2, b2, m)


def _sc_pool_body(vals_hbm, idx_hbm, out_vals, out_cnt,
                  rows_v, idx_v, idx_tail_v, ones_v, zb_v, zb16_v,
                  acc_vals, acc_cnt):
    cid = lax.axis_index("c")
    sid = lax.axis_index("s")
    wid = sid * 2 + cid

    zvec = jnp.zeros((16,), jnp.float32)
    onevec = jnp.where(lax.iota(jnp.int32, 16) == 0, 1.0, 0.0).astype(jnp.float32)

    def _zrow(r, _):
        for j in range(D_OUT // 16):
            zb_v[r, pl.ds(j * 16, 16)] = zvec
        zb16_v[r, :] = zvec
        return 0

    lax.fori_loop(0, SEG_PER_TILE, _zrow, 0)

    def _orow(r, _):
        ones_v[r, :] = onevec
        return 0

    lax.fori_loop(0, CHUNK, _orow, 0)

    pltpu.sync_copy(zb_v, acc_vals.at[pl.ds(sid * SEG_PER_TILE, SEG_PER_TILE)])
    pltpu.sync_copy(zb16_v, acc_cnt.at[pl.ds(sid * SEG_PER_TILE, SEG_PER_TILE)])
    plsc.subcore_barrier()

    def _chunk(k, _):
        ch = wid + k * NW

        @pl.when(ch < NCH_FULL)
        def _():
            base = ch * CHUNK
            pltpu.sync_copy(vals_hbm.at[pl.ds(base, CHUNK)], rows_v)
            pltpu.sync_copy(idx_hbm.at[pl.ds(base, CHUNK)], idx_v)
            pltpu.sync_copy(rows_v, acc_vals.at[idx_v], add=True)
            pltpu.sync_copy(ones_v, acc_cnt.at[idx_v], add=True)

        return 0

    lax.fori_loop(0, KMAX, _chunk, 0)

    @pl.when(wid == NW - 1)
    def _():
        base = NCH_FULL * CHUNK
        pltpu.sync_copy(vals_hbm.at[pl.ds(base, TAIL)], rows_v.at[pl.ds(0, TAIL)])
        pltpu.sync_copy(idx_hbm.at[pl.ds(base, TAIL)], idx_tail_v)
        pltpu.sync_copy(rows_v.at[pl.ds(0, TAIL)], acc_vals.at[idx_tail_v], add=True)
        pltpu.sync_copy(ones_v.at[pl.ds(0, TAIL)], acc_cnt.at[idx_tail_v], add=True)

    plsc.subcore_barrier()

    sl = pl.ds(sid * SEG_PER_TILE, SEG_PER_TILE)
    pltpu.sync_copy(acc_vals.at[sl], out_vals.at[cid, sl])
    pltpu.sync_copy(acc_cnt.at[sl], out_cnt.at[cid, sl])


def _sc_pool(vals, idx):
    mesh = plsc.VectorSubcoreMesh(core_axis_name="c", subcore_axis_name="s")
    f = pl.kernel(
        _sc_pool_body,
        out_type=(
            jax.ShapeDtypeStruct((2, NUM_SEG, D_OUT), jnp.float32),
            jax.ShapeDtypeStruct((2, NUM_SEG, 16), jnp.float32),
        ),
        mesh=mesh,
        scratch_types=[
            pltpu.VMEM((CHUNK, D_OUT), jnp.float32),
            pltpu.VMEM((CHUNK,), jnp.int32),
            pltpu.VMEM((TAIL,), jnp.int32),
            pltpu.VMEM((CHUNK, 16), jnp.float32),
            pltpu.VMEM((SEG_PER_TILE, D_OUT), jnp.float32),
            pltpu.VMEM((SEG_PER_TILE, 16), jnp.float32),
            pltpu.VMEM_SHARED((NUM_SEG, D_OUT), jnp.float32),
            pltpu.VMEM_SHARED((NUM_SEG, 16), jnp.float32),
        ],
    )
    return f(vals, idx)


def _merge_body(av_ref, ac_ref, zsum_ref, out_ref):
    v = av_ref[0] + av_ref[1]
    c = ac_ref[0, :, 0:1] + ac_ref[1, :, 0:1]
    zt = zsum_ref[0]
    out_ref[...] = v / (zt * jnp.maximum(c, 1.0))


def _merge(acc_vals, acc_cnt, zsum):
    return pl.pallas_call(
        _merge_body,
        grid=(1,),
        in_specs=[
            pl.BlockSpec((2, NUM_SEG, D_OUT), lambda i: (0, 0, 0)),
            pl.BlockSpec((2, NUM_SEG, 16), lambda i: (0, 0, 0)),
            pl.BlockSpec(memory_space=pltpu.SMEM),
        ],
        out_specs=pl.BlockSpec((NUM_SEG, D_OUT), lambda i: (0, 0)),
        out_shape=jax.ShapeDtypeStruct((NUM_SEG, D_OUT), jnp.float32),
    )(acc_vals, acc_cnt, zsum)


def kernel(z, batch_index, W1, b1, W2, b2):
    seg = batch_index.astype(jnp.int32)
    b1r = b1.reshape(1, HID)
    m = _scores_max(z, W1, b1r, W2)
    vals, zsum = _weighted_vals(z, W1, b1r, W2, b2.reshape(1), m.reshape(1))
    acc_vals, acc_cnt = _sc_pool(vals, seg)
    return _merge(acc_vals, acc_cnt, zsum.reshape(1))


# speedup vs baseline: 1.7924x; 1.7924x over previous
import functools

import jax
import jax.numpy as jnp
from jax import lax
from jax.experimental import pallas as pl
from jax.experimental.pallas import tpu as pltpu
from jax.experimental.pallas import tpu_sc as plsc

N = 100000
D_IN = 144
D_OUT = 128
HID = 64
NUM_SEG = 1024

RBLK = 2000
NBLK = N // RBLK

CHUNK = 128
NCH_FULL = N // CHUNK
TAIL = N - NCH_FULL * CHUNK
NW = 32
KMAX = (NCH_FULL + NW - 1) // NW
SEG_PER_TILE = NUM_SEG // 16


def _max_body(z_ref, w1_ref, b1_ref, w2_ref, m_ref, m_acc):
    i = pl.program_id(0)
    h = jnp.maximum(
        jnp.dot(z_ref[...], w1_ref[...], preferred_element_type=jnp.float32)
        + b1_ref[...],
        0.0,
    )
    s = jnp.dot(h, w2_ref[...], preferred_element_type=jnp.float32)
    bm = jnp.max(s)

    @pl.when(i == 0)
    def _():
        m_acc[0] = bm

    @pl.when(i > 0)
    def _():
        m_acc[0] = jnp.maximum(m_acc[0], bm)

    @pl.when(i == NBLK - 1)
    def _():
        m_ref[0, 0] = m_acc[0]


def _scores_max(z, W1, b1, W2):
    return pl.pallas_call(
        _max_body,
        grid=(NBLK,),
        in_specs=[
            pl.BlockSpec((RBLK, D_IN), lambda i: (i, 0)),
            pl.BlockSpec((D_IN, HID), lambda i: (0, 0)),
            pl.BlockSpec((1, HID), lambda i: (0, 0)),
            pl.BlockSpec((HID, 1), lambda i: (0, 0)),
        ],
        out_specs=pl.BlockSpec(memory_space=pltpu.SMEM),
        out_shape=jax.ShapeDtypeStruct((1, 1), jnp.float32),
        scratch_shapes=[pltpu.SMEM((1,), jnp.float32)],
    )(z, W1, b1, W2)


def _vals_body(z_ref, zl_ref, w1_ref, b1_ref, w2_ref, b2_ref, m_ref,
               vals_ref, zsum_ref, z_acc):
    i = pl.program_id(0)
    h = jnp.maximum(
        jnp.dot(z_ref[...], w1_ref[...], preferred_element_type=jnp.float32)
        + b1_ref[...],
        0.0,
    )
    s = jnp.dot(h, w2_ref[...], preferred_element_type=jnp.float32) + b2_ref[0]
    e = jnp.exp(s - m_ref[0])
    vals_ref[...] = e * zl_ref[...]
    bs = jnp.sum(e)

    @pl.when(i == 0)
    def _():
        z_acc[0] = bs

    @pl.when(i > 0)
    def _():
        z_acc[0] = z_acc[0] + bs

    @pl.when(i == NBLK - 1)
    def _():
        zsum_ref[0, 0] = z_acc[0]


def _weighted_vals(z, W1, b1, W2, b2, m):
    return pl.pallas_call(
        _vals_body,
        grid=(NBLK,),
        in_specs=[
            pl.BlockSpec((RBLK, D_IN), lambda i: (i, 0)),
            pl.BlockSpec((RBLK, D_OUT), lambda i: (i, 0)),
            pl.BlockSpec((D_IN, HID), lambda i: (0, 0)),
            pl.BlockSpec((1, HID), lambda i: (0, 0)),
            pl.BlockSpec((HID, 1), lambda i: (0, 0)),
            pl.BlockSpec(memory_space=pltpu.SMEM),
            pl.BlockSpec(memory_space=pltpu.SMEM),
        ],
        out_specs=[
            pl.BlockSpec((RBLK, D_OUT), lambda i: (i, 0)),
            pl.BlockSpec(memory_space=pltpu.SMEM),
        ],
        out_shape=[
            jax.ShapeDtypeStruct((N, D_OUT), jnp.float32),
            jax.ShapeDtypeStruct((1, 1), jnp.float32),
        ],
        scratch_shapes=[pltpu.SMEM((1,), jnp.float32)],
    )(z, z, W1, b1, W2, b2, m)


def _sc_pool_body(vals_hbm, idx_hbm, out_vals, out_cnt,
                  rows_v, idx_v, idx_tail_v, ones_v, zb_v,
                  acc_vals, acc_cnt):
    cid = lax.axis_index("c")
    sid = lax.axis_index("s")
    wid = sid * 2 + cid

    zvec = jnp.zeros((16,), jnp.float32)
    onevec = jnp.where(lax.iota(jnp.int32, 16) == 0, 1.0, 0.0).astype(jnp.float32)

    def _zrow(r, _):
        for j in range(D_OUT // 16):
            zb_v[r, pl.ds(j * 16, 16)] = zvec
        return 0

    lax.fori_loop(0, SEG_PER_TILE, _zrow, 0)

    def _orow(r, _):
        ones_v[r, pl.ds(0, 16)] = onevec
        for j in range(1, D_OUT // 16):
            ones_v[r, pl.ds(j * 16, 16)] = zvec
        return 0

    lax.fori_loop(0, CHUNK, _orow, 0)

    pltpu.sync_copy(zb_v, acc_vals.at[pl.ds(sid * SEG_PER_TILE, SEG_PER_TILE)])
    pltpu.sync_copy(zb_v, acc_cnt.at[pl.ds(sid * SEG_PER_TILE, SEG_PER_TILE)])
    plsc.subcore_barrier()

    def _chunk(k, _):
        ch = wid + k * NW

        @pl.when(ch < NCH_FULL)
        def _():
            base = ch * CHUNK
            pltpu.sync_copy(vals_hbm.at[pl.ds(base, CHUNK)], rows_v)
            pltpu.sync_copy(idx_hbm.at[pl.ds(base, CHUNK)], idx_v)
            pltpu.sync_copy(rows_v, acc_vals.at[idx_v], add=True)
            pltpu.sync_copy(ones_v, acc_cnt.at[idx_v], add=True)

        return 0

    lax.fori_loop(0, KMAX, _chunk, 0)

    @pl.when(wid == NW - 1)
    def _():
        base = NCH_FULL * CHUNK
        pltpu.sync_copy(vals_hbm.at[pl.ds(base, TAIL)], rows_v.at[pl.ds(0, TAIL)])
        pltpu.sync_copy(idx_hbm.at[pl.ds(base, TAIL)], idx_tail_v)
        pltpu.sync_copy(rows_v.at[pl.ds(0, TAIL)], acc_vals.at[idx_tail_v], add=True)
        pltpu.sync_copy(ones_v.at[pl.ds(0, TAIL)], acc_cnt.at[idx_tail_v], add=True)

    plsc.subcore_barrier()

    sl = pl.ds(sid * SEG_PER_TILE, SEG_PER_TILE)
    pltpu.sync_copy(acc_vals.at[sl], out_vals.at[cid, sl])
    pltpu.sync_copy(acc_cnt.at[sl], out_cnt.at[cid, sl])


def _sc_pool(vals, idx):
    mesh = plsc.VectorSubcoreMesh(core_axis_name="c", subcore_axis_name="s")
    f = pl.kernel(
        _sc_pool_body,
        out_type=(
            jax.ShapeDtypeStruct((2, NUM_SEG, D_OUT), jnp.float32),
            jax.ShapeDtypeStruct((2, NUM_SEG, D_OUT), jnp.float32),
        ),
        mesh=mesh,
        scratch_types=[
            pltpu.VMEM((CHUNK, D_OUT), jnp.float32),
            pltpu.VMEM((CHUNK,), jnp.int32),
            pltpu.VMEM((TAIL,), jnp.int32),
            pltpu.VMEM((CHUNK, D_OUT), jnp.float32),
            pltpu.VMEM((SEG_PER_TILE, D_OUT), jnp.float32),
            pltpu.VMEM_SHARED((NUM_SEG, D_OUT), jnp.float32),
            pltpu.VMEM_SHARED((NUM_SEG, D_OUT), jnp.float32),
        ],
    )
    return f(vals, idx)


def _merge_body(av_ref, ac_ref, zsum_ref, out_ref):
    v = av_ref[0] + av_ref[1]
    c = ac_ref[0, :, 0:1] + ac_ref[1, :, 0:1]
    zt = zsum_ref[0]
    out_ref[...] = v / (zt * jnp.maximum(c, 1.0))


def _merge(acc_vals, acc_cnt, zsum):
    return pl.pallas_call(
        _merge_body,
        grid=(1,),
        in_specs=[
            pl.BlockSpec((2, NUM_SEG, D_OUT), lambda i: (0, 0, 0)),
            pl.BlockSpec((2, NUM_SEG, D_OUT), lambda i: (0, 0, 0)),
            pl.BlockSpec(memory_space=pltpu.SMEM),
        ],
        out_specs=pl.BlockSpec((NUM_SEG, D_OUT), lambda i: (0, 0)),
        out_shape=jax.ShapeDtypeStruct((NUM_SEG, D_OUT), jnp.float32),
    )(acc_vals, acc_cnt, zsum)


def kernel(z, batch_index, W1, b1, W2, b2):
    seg = batch_index.astype(jnp.int32)
    b1r = b1.reshape(1, HID)
    m = _scores_max(z, W1, b1r, W2)
    vals, zsum = _weighted_vals(z, W1, b1r, W2, b2.reshape(1), m.reshape(1))
    acc_vals, acc_cnt = _sc_pool(vals, seg)
    return _merge(acc_vals, acc_cnt, zsum.reshape(1))


# speedup vs baseline: 1.8544x; 1.0346x over previous
import jax
import jax.numpy as jnp
from jax import lax
from jax.experimental import pallas as pl
from jax.experimental.pallas import tpu as pltpu
from jax.experimental.pallas import tpu_sc as plsc

N = 100000
D_IN = 144
D_OUT = 128
D_HI = D_IN - D_OUT
HID = 64
NUM_SEG = 1024

RBLK = 2000
NBLK = N // RBLK

CHUNK = 128
NCH_FULL = N // CHUNK
TAIL = N - NCH_FULL * CHUNK
NW = 32
KMAX = (NCH_FULL + NW - 1) // NW
SEG_PER_TILE = NUM_SEG // 16


def _max_body(za_ref, zb_ref, w1a_ref, w1b_ref, b1_ref, w2_ref, m_ref, m_acc):
    i = pl.program_id(0)
    h = jnp.maximum(
        jnp.dot(za_ref[...], w1a_ref[...], preferred_element_type=jnp.float32)
        + jnp.dot(zb_ref[...], w1b_ref[...], preferred_element_type=jnp.float32)
        + b1_ref[...],
        0.0,
    )
    s = jnp.dot(h, w2_ref[...], preferred_element_type=jnp.float32)
    bm = jnp.max(s)

    @pl.when(i == 0)
    def _():
        m_acc[0] = bm

    @pl.when(i > 0)
    def _():
        m_acc[0] = jnp.maximum(m_acc[0], bm)

    @pl.when(i == NBLK - 1)
    def _():
        m_ref[0, 0] = m_acc[0]


def _scores_max(za, zb, W1a, W1b, b1, W2):
    return pl.pallas_call(
        _max_body,
        grid=(NBLK,),
        in_specs=[
            pl.BlockSpec((RBLK, D_OUT), lambda i: (i, 0)),
            pl.BlockSpec((RBLK, D_HI), lambda i: (i, 0)),
            pl.BlockSpec((D_OUT, HID), lambda i: (0, 0)),
            pl.BlockSpec((D_HI, HID), lambda i: (0, 0)),
            pl.BlockSpec((1, HID), lambda i: (0, 0)),
            pl.BlockSpec((HID, 1), lambda i: (0, 0)),
        ],
        out_specs=pl.BlockSpec(memory_space=pltpu.SMEM),
        out_shape=jax.ShapeDtypeStruct((1, 1), jnp.float32),
        scratch_shapes=[pltpu.SMEM((1,), jnp.float32)],
    )(za, zb, W1a, W1b, b1, W2)


def _vals_body(za_ref, zb_ref, w1a_ref, w1b_ref, b1_ref, w2_ref, b2_ref, m_ref,
               vals_ref, zsum_ref, z_acc):
    i = pl.program_id(0)
    h = jnp.maximum(
        jnp.dot(za_ref[...], w1a_ref[...], preferred_element_type=jnp.float32)
        + jnp.dot(zb_ref[...], w1b_ref[...], preferred_element_type=jnp.float32)
        + b1_ref[...],
        0.0,
    )
    s = jnp.dot(h, w2_ref[...], preferred_element_type=jnp.float32) + b2_ref[0]
    e = jnp.exp(s - m_ref[0])
    vals_ref[...] = e * za_ref[...]
    bs = jnp.sum(e)

    @pl.when(i == 0)
    def _():
        z_acc[0] = bs

    @pl.when(i > 0)
    def _():
        z_acc[0] = z_acc[0] + bs

    @pl.when(i == NBLK - 1)
    def _():
        zsum_ref[0, 0] = z_acc[0]


def _weighted_vals(za, zb, W1a, W1b, b1, W2, b2, m):
    return pl.pallas_call(
        _vals_body,
        grid=(NBLK,),
        in_specs=[
            pl.BlockSpec((RBLK, D_OUT), lambda i: (i, 0)),
            pl.BlockSpec((RBLK, D_HI), lambda i: (i, 0)),
            pl.BlockSpec((D_OUT, HID), lambda i: (0, 0)),
            pl.BlockSpec((D_HI, HID), lambda i: (0, 0)),
            pl.BlockSpec((1, HID), lambda i: (0, 0)),
            pl.BlockSpec((HID, 1), lambda i: (0, 0)),
            pl.BlockSpec(memory_space=pltpu.SMEM),
            pl.BlockSpec(memory_space=pltpu.SMEM),
        ],
        out_specs=[
            pl.BlockSpec((RBLK, D_OUT), lambda i: (i, 0)),
            pl.BlockSpec(memory_space=pltpu.SMEM),
        ],
        out_shape=[
            jax.ShapeDtypeStruct((N, D_OUT), jnp.float32),
            jax.ShapeDtypeStruct((1, 1), jnp.float32),
        ],
        scratch_shapes=[pltpu.SMEM((1,), jnp.float32)],
    )(za, zb, W1a, W1b, b1, W2, b2, m)


def _sc_pool_body(vals_hbm, idx_hbm, out_vals, out_cnt,
                  rows_v, idx_v, idx_tail_v, ones_v, zb_v,
                  acc_vals, acc_cnt, gsem, csem):
    cid = lax.axis_index("c")
    sid = lax.axis_index("s")
    wid = sid * 2 + cid

    zvec = jnp.zeros((16,), jnp.float32)
    onevec = jnp.where(lax.iota(jnp.int32, 16) == 0, 1.0, 0.0).astype(jnp.float32)

    def _zrow(r, _):
        for j in range(D_OUT // 16):
            zb_v[r, pl.ds(j * 16, 16)] = zvec
        return 0

    lax.fori_loop(0, SEG_PER_TILE, _zrow, 0)

    def _orow(r, _):
        ones_v[r, pl.ds(0, 16)] = onevec
        for j in range(1, D_OUT // 16):
            ones_v[r, pl.ds(j * 16, 16)] = zvec
        return 0

    lax.fori_loop(0, CHUNK, _orow, 0)

    pltpu.sync_copy(zb_v, acc_vals.at[pl.ds(sid * SEG_PER_TILE, SEG_PER_TILE)])
    pltpu.sync_copy(zb_v, acc_cnt.at[pl.ds(sid * SEG_PER_TILE, SEG_PER_TILE)])
    plsc.subcore_barrier()

    def _start_gather(ch, b):
        base = ch * CHUNK
        pltpu.async_copy(vals_hbm.at[pl.ds(base, CHUNK)], rows_v.at[b], gsem.at[b])
        pltpu.async_copy(idx_hbm.at[pl.ds(base, CHUNK)], idx_v.at[b], gsem.at[b])

    def _wait_gather(ch, b):
        base = ch * CHUNK
        pltpu.make_async_copy(
            vals_hbm.at[pl.ds(base, CHUNK)], rows_v.at[b], gsem.at[b]).wait()
        pltpu.make_async_copy(
            idx_hbm.at[pl.ds(base, CHUNK)], idx_v.at[b], gsem.at[b]).wait()

    _start_gather(wid, 0)

    def _pair(kk, _):
        k2 = kk * 2
        for b in (0, 1):
            k = k2 + b
            ch = wid + k * NW
            nxt = ch + NW

            @pl.when(ch < NCH_FULL)
            def _():
                _wait_gather(ch, b)

                @pl.when(nxt < NCH_FULL)
                def _():
                    _start_gather(nxt, 1 - b)

                pltpu.sync_copy(rows_v.at[b], acc_vals.at[idx_v.at[b]], add=True)
                pltpu.sync_copy(ones_v, acc_cnt.at[idx_v.at[b]], add=True)

        return 0

    lax.fori_loop(0, (KMAX + 1) // 2, _pair, 0)

    @pl.when(wid == NW - 1)
    def _():
        base = NCH_FULL * CHUNK
        pltpu.sync_copy(vals_hbm.at[pl.ds(base, TAIL)],
                        rows_v.at[0].at[pl.ds(0, TAIL)])
        pltpu.sync_copy(idx_hbm.at[pl.ds(base, TAIL)], idx_tail_v)
        pltpu.sync_copy(rows_v.at[0].at[pl.ds(0, TAIL)],
                        acc_vals.at[idx_tail_v], add=True)
        pltpu.sync_copy(ones_v.at[pl.ds(0, TAIL)],
                        acc_cnt.at[idx_tail_v], add=True)

    plsc.subcore_barrier()

    sl = pl.ds(sid * SEG_PER_TILE, SEG_PER_TILE)
    pltpu.sync_copy(acc_vals.at[sl], out_vals.at[cid, sl])
    pltpu.sync_copy(acc_cnt.at[sl], out_cnt.at[cid, sl])


def _sc_pool(vals, idx):
    mesh = plsc.VectorSubcoreMesh(core_axis_name="c", subcore_axis_name="s")
    f = pl.kernel(
        _sc_pool_body,
        out_type=(
            jax.ShapeDtypeStruct((2, NUM_SEG, D_OUT), jnp.float32),
            jax.ShapeDtypeStruct((2, NUM_SEG, D_OUT), jnp.float32),
        ),
        mesh=mesh,
        scratch_types=[
            pltpu.VMEM((2, CHUNK, D_OUT), jnp.float32),
            pltpu.VMEM((2, CHUNK), jnp.int32),
            pltpu.VMEM((TAIL,), jnp.int32),
            pltpu.VMEM((CHUNK, D_OUT), jnp.float32),
            pltpu.VMEM((SEG_PER_TILE, D_OUT), jnp.float32),
            pltpu.VMEM_SHARED((NUM_SEG, D_OUT), jnp.float32),
            pltpu.VMEM_SHARED((NUM_SEG, D_OUT), jnp.float32),
            pltpu.SemaphoreType.DMA((2,)),
            pltpu.SemaphoreType.DMA,
        ],
    )
    return f(vals, idx)


def _merge_body(av_ref, ac_ref, zsum_ref, out_ref):
    v = av_ref[0] + av_ref[1]
    c = ac_ref[0, :, 0:1] + ac_ref[1, :, 0:1]
    zt = zsum_ref[0]
    out_ref[...] = v / (zt * jnp.maximum(c, 1.0))


def _merge(acc_vals, acc_cnt, zsum):
    return pl.pallas_call(
        _merge_body,
        grid=(1,),
        in_specs=[
            pl.BlockSpec((2, NUM_SEG, D_OUT), lambda i: (0, 0, 0)),
            pl.BlockSpec((2, NUM_SEG, D_OUT), lambda i: (0, 0, 0)),
            pl.BlockSpec(memory_space=pltpu.SMEM),
        ],
        out_specs=pl.BlockSpec((NUM_SEG, D_OUT), lambda i: (0, 0)),
        out_shape=jax.ShapeDtypeStruct((NUM_SEG, D_OUT), jnp.float32),
    )(acc_vals, acc_cnt, zsum)


def kernel(z, batch_index, W1, b1, W2, b2):
    seg = batch_index.astype(jnp.int32)
    za = z[:, :D_OUT]
    zb = z[:, D_OUT:]
    W1a = W1[:D_OUT]
    W1b = W1[D_OUT:]
    b1r = b1.reshape(1, HID)
    m = _scores_max(za, zb, W1a, W1b, b1r, W2)
    vals, zsum = _weighted_vals(za, zb, W1a, W1b, b1r, W2, b2.reshape(1),
                                m.reshape(1))
    acc_vals, acc_cnt = _sc_pool(vals, seg)
    return _merge(acc_vals, acc_cnt, zsum.reshape(1))


# speedup vs baseline: 1.8606x; 1.0034x over previous
import jax
import jax.numpy as jnp
from jax import lax
from jax.experimental import pallas as pl
from jax.experimental.pallas import tpu as pltpu
from jax.experimental.pallas import tpu_sc as plsc

N = 100000
D_IN = 144
D_OUT = 128
D_HI = D_IN - D_OUT
HID = 64
NUM_SEG = 1024

RBLK = 2000
NBLK = N // RBLK

CHUNK = 128
NCH_FULL = N // CHUNK
TAIL = N - NCH_FULL * CHUNK
NW = 32
KMAX = (NCH_FULL + NW - 1) // NW
SEG_PER_TILE = NUM_SEG // 16


def _max_body(za_ref, zb_ref, w1a_ref, w1b_ref, b1_ref, w2_ref, m_ref, m_acc):
    i = pl.program_id(0)
    h = jnp.maximum(
        jnp.dot(za_ref[...], w1a_ref[...], preferred_element_type=jnp.float32)
        + jnp.dot(zb_ref[...], w1b_ref[...], preferred_element_type=jnp.float32)
        + b1_ref[...],
        0.0,
    )
    s = jnp.dot(h, w2_ref[...], preferred_element_type=jnp.float32)
    bm = jnp.max(s)

    @pl.when(i == 0)
    def _():
        m_acc[0] = bm

    @pl.when(i > 0)
    def _():
        m_acc[0] = jnp.maximum(m_acc[0], bm)

    @pl.when(i == NBLK - 1)
    def _():
        m_ref[0, 0] = m_acc[0]


def _scores_max(za, zb, W1a, W1b, b1, W2):
    return pl.pallas_call(
        _max_body,
        grid=(NBLK,),
        in_specs=[
            pl.BlockSpec((RBLK, D_OUT), lambda i: (i, 0)),
            pl.BlockSpec((RBLK, D_HI), lambda i: (i, 0)),
            pl.BlockSpec((D_OUT, HID), lambda i: (0, 0)),
            pl.BlockSpec((D_HI, HID), lambda i: (0, 0)),
            pl.BlockSpec((1, HID), lambda i: (0, 0)),
            pl.BlockSpec((HID, 1), lambda i: (0, 0)),
        ],
        out_specs=pl.BlockSpec(memory_space=pltpu.SMEM),
        out_shape=jax.ShapeDtypeStruct((1, 1), jnp.float32),
        scratch_shapes=[pltpu.SMEM((1,), jnp.float32)],
    )(za, zb, W1a, W1b, b1, W2)


def _vals_body(za_ref, zb_ref, w1a_ref, w1b_ref, b1_ref, w2_ref, b2_ref, m_ref,
               vals_ref, zsum_ref, z_acc):
    i = pl.program_id(0)
    h = jnp.maximum(
        jnp.dot(za_ref[...], w1a_ref[...], preferred_element_type=jnp.float32)
        + jnp.dot(zb_ref[...], w1b_ref[...], preferred_element_type=jnp.float32)
        + b1_ref[...],
        0.0,
    )
    s = jnp.dot(h, w2_ref[...], preferred_element_type=jnp.float32) + b2_ref[0]
    e = jnp.exp(s - m_ref[0])
    vals_ref[...] = e * za_ref[...]
    bs = jnp.sum(e)

    @pl.when(i == 0)
    def _():
        z_acc[0] = bs

    @pl.when(i > 0)
    def _():
        z_acc[0] = z_acc[0] + bs

    @pl.when(i == NBLK - 1)
    def _():
        zsum_ref[0, 0] = z_acc[0]


def _weighted_vals(za, zb, W1a, W1b, b1, W2, b2, m):
    return pl.pallas_call(
        _vals_body,
        grid=(NBLK,),
        in_specs=[
            pl.BlockSpec((RBLK, D_OUT), lambda i: (i, 0)),
            pl.BlockSpec((RBLK, D_HI), lambda i: (i, 0)),
            pl.BlockSpec((D_OUT, HID), lambda i: (0, 0)),
            pl.BlockSpec((D_HI, HID), lambda i: (0, 0)),
            pl.BlockSpec((1, HID), lambda i: (0, 0)),
            pl.BlockSpec((HID, 1), lambda i: (0, 0)),
            pl.BlockSpec(memory_space=pltpu.SMEM),
            pl.BlockSpec(memory_space=pltpu.SMEM),
        ],
        out_specs=[
            pl.BlockSpec((RBLK, D_OUT), lambda i: (i, 0)),
            pl.BlockSpec(memory_space=pltpu.SMEM),
        ],
        out_shape=[
            jax.ShapeDtypeStruct((N, D_OUT), jnp.float32),
            jax.ShapeDtypeStruct((1, 1), jnp.float32),
        ],
        scratch_shapes=[pltpu.SMEM((1,), jnp.float32)],
    )(za, zb, W1a, W1b, b1, W2, b2, m)


def _sc_pool_body(vals_hbm, idx_hbm, out_vals, out_cnt,
                  rows_v, idx_v, idx_tail_v, ones_v, zb_v,
                  acc_vals, acc_cnt, gsem, csem):
    cid = lax.axis_index("c")
    sid = lax.axis_index("s")
    wid = sid * 2 + cid

    zvec = jnp.zeros((16,), jnp.float32)
    onevec = jnp.where(lax.iota(jnp.int32, 16) == 0, 1.0, 0.0).astype(jnp.float32)

    def _zrow(r, _):
        for j in range(D_OUT // 16):
            zb_v[r, pl.ds(j * 16, 16)] = zvec
        return 0

    lax.fori_loop(0, SEG_PER_TILE, _zrow, 0)

    def _orow(r, _):
        ones_v[r, pl.ds(0, 16)] = onevec
        for j in range(1, D_OUT // 16):
            ones_v[r, pl.ds(j * 16, 16)] = zvec
        return 0

    lax.fori_loop(0, CHUNK, _orow, 0)

    pltpu.sync_copy(zb_v, acc_vals.at[pl.ds(sid * SEG_PER_TILE, SEG_PER_TILE)])
    pltpu.sync_copy(zb_v, acc_cnt.at[pl.ds(sid * SEG_PER_TILE, SEG_PER_TILE)])
    plsc.subcore_barrier()

    def _start_gather(ch, b):
        base = ch * CHUNK
        pltpu.async_copy(vals_hbm.at[pl.ds(base, CHUNK)], rows_v.at[b], gsem.at[b])
        pltpu.async_copy(idx_hbm.at[pl.ds(base, CHUNK)], idx_v.at[b], gsem.at[b])

    def _wait_gather(ch, b):
        base = ch * CHUNK
        pltpu.make_async_copy(
            vals_hbm.at[pl.ds(base, CHUNK)], rows_v.at[b], gsem.at[b]).wait()
        pltpu.make_async_copy(
            idx_hbm.at[pl.ds(base, CHUNK)], idx_v.at[b], gsem.at[b]).wait()

    _start_gather(wid, 0)

    def _pair(kk, _):
        k2 = kk * 2
        for b in (0, 1):
            k = k2 + b
            ch = wid + k * NW
            nxt = ch + NW

            @pl.when(ch < NCH_FULL)
            def _():
                _wait_gather(ch, b)

                @pl.when(nxt < NCH_FULL)
                def _():
                    _start_gather(nxt, 1 - b)

                pltpu.async_copy(ones_v, acc_cnt.at[idx_v.at[b]], csem, add=True)
                pltpu.sync_copy(rows_v.at[b], acc_vals.at[idx_v.at[b]], add=True)
                pltpu.make_async_copy(
                    ones_v, acc_cnt.at[idx_v.at[b]], csem).wait()

        return 0

    lax.fori_loop(0, (KMAX + 1) // 2, _pair, 0)

    @pl.when(wid == NW - 1)
    def _():
        base = NCH_FULL * CHUNK
        pltpu.sync_copy(vals_hbm.at[pl.ds(base, TAIL)],
                        rows_v.at[0].at[pl.ds(0, TAIL)])
        pltpu.sync_copy(idx_hbm.at[pl.ds(base, TAIL)], idx_tail_v)
        pltpu.sync_copy(rows_v.at[0].at[pl.ds(0, TAIL)],
                        acc_vals.at[idx_tail_v], add=True)
        pltpu.sync_copy(ones_v.at[pl.ds(0, TAIL)],
                        acc_cnt.at[idx_tail_v], add=True)

    plsc.subcore_barrier()

    sl = pl.ds(sid * SEG_PER_TILE, SEG_PER_TILE)
    pltpu.sync_copy(acc_vals.at[sl], out_vals.at[cid, sl])
    pltpu.sync_copy(acc_cnt.at[sl], out_cnt.at[cid, sl])


def _sc_pool(vals, idx):
    mesh = plsc.VectorSubcoreMesh(core_axis_name="c", subcore_axis_name="s")
    f = pl.kernel(
        _sc_pool_body,
        out_type=(
            jax.ShapeDtypeStruct((2, NUM_SEG, D_OUT), jnp.float32),
            jax.ShapeDtypeStruct((2, NUM_SEG, D_OUT), jnp.float32),
        ),
        mesh=mesh,
        scratch_types=[
            pltpu.VMEM((2, CHUNK, D_OUT), jnp.float32),
            pltpu.VMEM((2, CHUNK), jnp.int32),
            pltpu.VMEM((TAIL,), jnp.int32),
            pltpu.VMEM((CHUNK, D_OUT), jnp.float32),
            pltpu.VMEM((SEG_PER_TILE, D_OUT), jnp.float32),
            pltpu.VMEM_SHARED((NUM_SEG, D_OUT), jnp.float32),
            pltpu.VMEM_SHARED((NUM_SEG, D_OUT), jnp.float32),
            pltpu.SemaphoreType.DMA((2,)),
            pltpu.SemaphoreType.DMA,
        ],
    )
    return f(vals, idx)


def _merge_body(av_ref, ac_ref, zsum_ref, out_ref):
    v = av_ref[0] + av_ref[1]
    c = ac_ref[0, :, 0:1] + ac_ref[1, :, 0:1]
    zt = zsum_ref[0]
    out_ref[...] = v / (zt * jnp.maximum(c, 1.0))


def _merge(acc_vals, acc_cnt, zsum):
    return pl.pallas_call(
        _merge_body,
        grid=(1,),
        in_specs=[
            pl.BlockSpec((2, NUM_SEG, D_OUT), lambda i: (0, 0, 0)),
            pl.BlockSpec((2, NUM_SEG, D_OUT), lambda i: (0, 0, 0)),
            pl.BlockSpec(memory_space=pltpu.SMEM),
        ],
        out_specs=pl.BlockSpec((NUM_SEG, D_OUT), lambda i: (0, 0)),
        out_shape=jax.ShapeDtypeStruct((NUM_SEG, D_OUT), jnp.float32),
    )(acc_vals, acc_cnt, zsum)


def kernel(z, batch_index, W1, b1, W2, b2):
    seg = batch_index.astype(jnp.int32)
    za = z[:, :D_OUT]
    zb = z[:, D_OUT:]
    W1a = W1[:D_OUT]
    W1b = W1[D_OUT:]
    b1r = b1.reshape(1, HID)
    m = _scores_max(za, zb, W1a, W1b, b1r, W2)
    vals, zsum = _weighted_vals(za, zb, W1a, W1b, b1r, W2, b2.reshape(1),
                                m.reshape(1))
    acc_vals, acc_cnt = _sc_pool(vals, seg)
    return _merge(acc_vals, acc_cnt, zsum.reshape(1))


# speedup vs baseline: 2.3670x; 1.2721x over previous
import jax
import jax.numpy as jnp
from jax import lax
from jax.experimental import pallas as pl
from jax.experimental.pallas import tpu as pltpu
from jax.experimental.pallas import tpu_sc as plsc

N = 100000
D_IN = 144
D_OUT = 128
D_HI = D_IN - D_OUT
HID = 64
NUM_SEG = 1024

RBLK = 4000
NBLK = N // RBLK

CHUNK = 128
NCH_FULL = N // CHUNK
TAIL = N - NCH_FULL * CHUNK
NW = 32
KMAX = (NCH_FULL + NW - 1) // NW
SEG_PER_TILE = NUM_SEG // 16


def _max_body(z_ref, w1_ref, b1_ref, w2_ref, m_ref, m_acc):
    i = pl.program_id(0)
    h = jnp.maximum(
        jnp.dot(z_ref[...], w1_ref[...], preferred_element_type=jnp.float32)
        + b1_ref[...],
        0.0,
    )
    s = jnp.dot(h, w2_ref[...], preferred_element_type=jnp.float32)
    bm = jnp.max(s)

    @pl.when(i == 0)
    def _():
        m_acc[0] = bm

    @pl.when(i > 0)
    def _():
        m_acc[0] = jnp.maximum(m_acc[0], bm)

    @pl.when(i == NBLK - 1)
    def _():
        m_ref[0, 0] = m_acc[0]


def _scores_max(z, W1, b1, W2):
    return pl.pallas_call(
        _max_body,
        grid=(NBLK,),
        in_specs=[
            pl.BlockSpec((RBLK, D_IN), lambda i: (i, 0)),
            pl.BlockSpec((D_IN, HID), lambda i: (0, 0)),
            pl.BlockSpec((1, HID), lambda i: (0, 0)),
            pl.BlockSpec((HID, 1), lambda i: (0, 0)),
        ],
        out_specs=pl.BlockSpec(memory_space=pltpu.SMEM),
        out_shape=jax.ShapeDtypeStruct((1, 1), jnp.float32),
        scratch_shapes=[pltpu.SMEM((1,), jnp.float32)],
    )(z, W1, b1, W2)


def _vals_body(z_ref, w1_ref, b1_ref, w2_ref, b2_ref, m_ref,
               vals_ref, zsum_ref, z_acc):
    i = pl.program_id(0)
    h = jnp.maximum(
        jnp.dot(z_ref[...], w1_ref[...], preferred_element_type=jnp.float32)
        + b1_ref[...],
        0.0,
    )
    s = jnp.dot(h, w2_ref[...], preferred_element_type=jnp.float32) + b2_ref[0]
    e = jnp.exp(s - m_ref[0])
    vals_ref[...] = e * z_ref[:, :D_OUT]
    bs = jnp.sum(e)

    @pl.when(i == 0)
    def _():
        z_acc[0] = bs

    @pl.when(i > 0)
    def _():
        z_acc[0] = z_acc[0] + bs

    @pl.when(i == NBLK - 1)
    def _():
        zsum_ref[0, 0] = z_acc[0]


def _weighted_vals(z, W1, b1, W2, b2, m):
    return pl.pallas_call(
        _vals_body,
        grid=(NBLK,),
        in_specs=[
            pl.BlockSpec((RBLK, D_IN), lambda i: (i, 0)),
            pl.BlockSpec((D_IN, HID), lambda i: (0, 0)),
            pl.BlockSpec((1, HID), lambda i: (0, 0)),
            pl.BlockSpec((HID, 1), lambda i: (0, 0)),
            pl.BlockSpec(memory_space=pltpu.SMEM),
            pl.BlockSpec(memory_space=pltpu.SMEM),
        ],
        out_specs=[
            pl.BlockSpec((RBLK, D_OUT), lambda i: (i, 0)),
            pl.BlockSpec(memory_space=pltpu.SMEM),
        ],
        out_shape=[
            jax.ShapeDtypeStruct((N, D_OUT), jnp.float32),
            jax.ShapeDtypeStruct((1, 1), jnp.float32),
        ],
        scratch_shapes=[pltpu.SMEM((1,), jnp.float32)],
    )(z, W1, b1, W2, b2, m)


def _sc_pool_body(vals_hbm, idx_hbm, out_vals, out_cnt,
                  rows_v, idx_v, idx_tail_v, ones_v, zb_v,
                  acc_vals, acc_cnt, gsem, csem):
    cid = lax.axis_index("c")
    sid = lax.axis_index("s")
    wid = sid * 2 + cid

    zvec = jnp.zeros((16,), jnp.float32)
    onevec = jnp.where(lax.iota(jnp.int32, 16) == 0, 1.0, 0.0).astype(jnp.float32)

    def _zrow(r, _):
        for j in range(D_OUT // 16):
            zb_v[r, pl.ds(j * 16, 16)] = zvec
        return 0

    lax.fori_loop(0, SEG_PER_TILE, _zrow, 0)

    def _orow(r, _):
        ones_v[r, pl.ds(0, 16)] = onevec
        for j in range(1, D_OUT // 16):
            ones_v[r, pl.ds(j * 16, 16)] = zvec
        return 0

    lax.fori_loop(0, CHUNK, _orow, 0)

    pltpu.sync_copy(zb_v, acc_vals.at[pl.ds(sid * SEG_PER_TILE, SEG_PER_TILE)])
    pltpu.sync_copy(zb_v, acc_cnt.at[pl.ds(sid * SEG_PER_TILE, SEG_PER_TILE)])
    plsc.subcore_barrier()

    def _start_gather(ch, b):
        base = ch * CHUNK
        pltpu.async_copy(vals_hbm.at[pl.ds(base, CHUNK)], rows_v.at[b], gsem.at[b])
        pltpu.async_copy(idx_hbm.at[pl.ds(base, CHUNK)], idx_v.at[b], gsem.at[b])

    def _wait_gather(ch, b):
        base = ch * CHUNK
        pltpu.make_async_copy(
            vals_hbm.at[pl.ds(base, CHUNK)], rows_v.at[b], gsem.at[b]).wait()
        pltpu.make_async_copy(
            idx_hbm.at[pl.ds(base, CHUNK)], idx_v.at[b], gsem.at[b]).wait()

    _start_gather(wid, 0)

    def _pair(kk, _):
        k2 = kk * 2
        for b in (0, 1):
            k = k2 + b
            ch = wid + k * NW
            nxt = ch + NW

            @pl.when(ch < NCH_FULL)
            def _():
                _wait_gather(ch, b)

                @pl.when(nxt < NCH_FULL)
                def _():
                    _start_gather(nxt, 1 - b)

                pltpu.async_copy(ones_v, acc_cnt.at[idx_v.at[b]], csem, add=True)
                pltpu.sync_copy(rows_v.at[b], acc_vals.at[idx_v.at[b]], add=True)
                pltpu.make_async_copy(
                    ones_v, acc_cnt.at[idx_v.at[b]], csem).wait()

        return 0

    lax.fori_loop(0, (KMAX + 1) // 2, _pair, 0)

    @pl.when(wid == NW - 1)
    def _():
        base = NCH_FULL * CHUNK
        pltpu.sync_copy(vals_hbm.at[pl.ds(base, TAIL)],
                        rows_v.at[0].at[pl.ds(0, TAIL)])
        pltpu.sync_copy(idx_hbm.at[pl.ds(base, TAIL)], idx_tail_v)
        pltpu.sync_copy(rows_v.at[0].at[pl.ds(0, TAIL)],
                        acc_vals.at[idx_tail_v], add=True)
        pltpu.sync_copy(ones_v.at[pl.ds(0, TAIL)],
                        acc_cnt.at[idx_tail_v], add=True)

    plsc.subcore_barrier()

    sl = pl.ds(sid * SEG_PER_TILE, SEG_PER_TILE)
    pltpu.sync_copy(acc_vals.at[sl], out_vals.at[cid, sl])
    pltpu.sync_copy(acc_cnt.at[sl], out_cnt.at[cid, sl])


def _sc_pool(vals, idx):
    mesh = plsc.VectorSubcoreMesh(core_axis_name="c", subcore_axis_name="s")
    f = pl.kernel(
        _sc_pool_body,
        out_type=(
            jax.ShapeDtypeStruct((2, NUM_SEG, D_OUT), jnp.float32),
            jax.ShapeDtypeStruct((2, NUM_SEG, D_OUT), jnp.float32),
        ),
        mesh=mesh,
        scratch_types=[
            pltpu.VMEM((2, CHUNK, D_OUT), jnp.float32),
            pltpu.VMEM((2, CHUNK), jnp.int32),
            pltpu.VMEM((TAIL,), jnp.int32),
            pltpu.VMEM((CHUNK, D_OUT), jnp.float32),
            pltpu.VMEM((SEG_PER_TILE, D_OUT), jnp.float32),
            pltpu.VMEM_SHARED((NUM_SEG, D_OUT), jnp.float32),
            pltpu.VMEM_SHARED((NUM_SEG, D_OUT), jnp.float32),
            pltpu.SemaphoreType.DMA((2,)),
            pltpu.SemaphoreType.DMA,
        ],
    )
    return f(vals, idx)


def _merge_body(av_ref, ac_ref, zsum_ref, out_ref):
    v = av_ref[0] + av_ref[1]
    c = ac_ref[0, :, 0:1] + ac_ref[1, :, 0:1]
    zt = zsum_ref[0]
    out_ref[...] = v / (zt * jnp.maximum(c, 1.0))


def _merge(acc_vals, acc_cnt, zsum):
    return pl.pallas_call(
        _merge_body,
        grid=(1,),
        in_specs=[
            pl.BlockSpec((2, NUM_SEG, D_OUT), lambda i: (0, 0, 0)),
            pl.BlockSpec((2, NUM_SEG, D_OUT), lambda i: (0, 0, 0)),
            pl.BlockSpec(memory_space=pltpu.SMEM),
        ],
        out_specs=pl.BlockSpec((NUM_SEG, D_OUT), lambda i: (0, 0)),
        out_shape=jax.ShapeDtypeStruct((NUM_SEG, D_OUT), jnp.float32),
    )(acc_vals, acc_cnt, zsum)


def kernel(z, batch_index, W1, b1, W2, b2):
    seg = batch_index.astype(jnp.int32)
    b1r = b1.reshape(1, HID)
    m = _scores_max(z, W1, b1r, W2)
    vals, zsum = _weighted_vals(z, W1, b1r, W2, b2.reshape(1), m.reshape(1))
    acc_vals, acc_cnt = _sc_pool(vals, seg)
    return _merge(acc_vals, acc_cnt, zsum.reshape(1))


# speedup vs baseline: 2.9382x; 1.2413x over previous
import jax
import jax.numpy as jnp
from jax import lax
from jax.experimental import pallas as pl
from jax.experimental.pallas import tpu as pltpu
from jax.experimental.pallas import tpu_sc as plsc

N = 100000
D_IN = 144
D_OUT = 128
D_HI = D_IN - D_OUT
HID = 64
NUM_SEG = 1024

RBLK = 10000
NBLK = N // RBLK

CHUNK = 128
NCH_FULL = N // CHUNK
TAIL = N - NCH_FULL * CHUNK
NW = 32
KMAX = (NCH_FULL + NW - 1) // NW
SEG_PER_TILE = NUM_SEG // 16


def _max_body(z_ref, w1_ref, b1_ref, w2_ref, m_ref, m_acc):
    i = pl.program_id(0)
    h = jnp.maximum(
        jnp.dot(z_ref[...], w1_ref[...], preferred_element_type=jnp.float32)
        + b1_ref[...],
        0.0,
    )
    s = jnp.dot(h, w2_ref[...], preferred_element_type=jnp.float32)
    bm = jnp.max(s)

    @pl.when(i == 0)
    def _():
        m_acc[0] = bm

    @pl.when(i > 0)
    def _():
        m_acc[0] = jnp.maximum(m_acc[0], bm)

    @pl.when(i == NBLK - 1)
    def _():
        m_ref[0, 0] = m_acc[0]


def _scores_max(z, W1, b1, W2):
    return pl.pallas_call(
        _max_body,
        grid=(NBLK,),
        in_specs=[
            pl.BlockSpec((RBLK, D_IN), lambda i: (i, 0)),
            pl.BlockSpec((D_IN, HID), lambda i: (0, 0)),
            pl.BlockSpec((1, HID), lambda i: (0, 0)),
            pl.BlockSpec((HID, 1), lambda i: (0, 0)),
        ],
        out_specs=pl.BlockSpec(memory_space=pltpu.SMEM),
        out_shape=jax.ShapeDtypeStruct((1, 1), jnp.float32),
        scratch_shapes=[pltpu.SMEM((1,), jnp.float32)],
    )(z, W1, b1, W2)


def _vals_body(z_ref, w1_ref, b1_ref, w2_ref, b2_ref, m_ref,
               vals_ref, zsum_ref, z_acc):
    i = pl.program_id(0)
    h = jnp.maximum(
        jnp.dot(z_ref[...], w1_ref[...], preferred_element_type=jnp.float32)
        + b1_ref[...],
        0.0,
    )
    s = jnp.dot(h, w2_ref[...], preferred_element_type=jnp.float32) + b2_ref[0]
    e = jnp.exp(s - m_ref[0])
    vals_ref[...] = e * z_ref[:, :D_OUT].astype(jnp.float32)
    bs = jnp.sum(e)

    @pl.when(i == 0)
    def _():
        z_acc[0] = bs

    @pl.when(i > 0)
    def _():
        z_acc[0] = z_acc[0] + bs

    @pl.when(i == NBLK - 1)
    def _():
        zsum_ref[0, 0] = z_acc[0]


def _weighted_vals(z, W1, b1, W2, b2, m):
    return pl.pallas_call(
        _vals_body,
        grid=(NBLK,),
        in_specs=[
            pl.BlockSpec((RBLK, D_IN), lambda i: (i, 0)),
            pl.BlockSpec((D_IN, HID), lambda i: (0, 0)),
            pl.BlockSpec((1, HID), lambda i: (0, 0)),
            pl.BlockSpec((HID, 1), lambda i: (0, 0)),
            pl.BlockSpec(memory_space=pltpu.SMEM),
            pl.BlockSpec(memory_space=pltpu.SMEM),
        ],
        out_specs=[
            pl.BlockSpec((RBLK, D_OUT), lambda i: (i, 0)),
            pl.BlockSpec(memory_space=pltpu.SMEM),
        ],
        out_shape=[
            jax.ShapeDtypeStruct((N, D_OUT), jnp.float32),
            jax.ShapeDtypeStruct((1, 1), jnp.float32),
        ],
        scratch_shapes=[pltpu.SMEM((1,), jnp.float32)],
    )(z, W1, b1, W2, b2, m)


def _sc_pool_body(vals_hbm, idx_hbm, out_vals, out_cnt,
                  rows_v, idx_v, idx_tail_v, ones_v, zb_v,
                  acc_vals, acc_cnt, gsem, csem):
    cid = lax.axis_index("c")
    sid = lax.axis_index("s")
    wid = sid * 2 + cid

    zvec = jnp.zeros((16,), jnp.float32)
    onevec = jnp.where(lax.iota(jnp.int32, 16) == 0, 1.0, 0.0).astype(jnp.float32)

    def _zrow(r, _):
        for j in range(D_OUT // 16):
            zb_v[r, pl.ds(j * 16, 16)] = zvec
        return 0

    lax.fori_loop(0, SEG_PER_TILE, _zrow, 0)

    def _orow(r, _):
        ones_v[r, pl.ds(0, 16)] = onevec
        for j in range(1, D_OUT // 16):
            ones_v[r, pl.ds(j * 16, 16)] = zvec
        return 0

    lax.fori_loop(0, CHUNK, _orow, 0)

    pltpu.sync_copy(zb_v, acc_vals.at[pl.ds(sid * SEG_PER_TILE, SEG_PER_TILE)])
    pltpu.sync_copy(zb_v, acc_cnt.at[pl.ds(sid * SEG_PER_TILE, SEG_PER_TILE)])
    plsc.subcore_barrier()

    def _start_gather(ch, b):
        base = ch * CHUNK
        pltpu.async_copy(vals_hbm.at[pl.ds(base, CHUNK)], rows_v.at[b], gsem.at[b])
        pltpu.async_copy(idx_hbm.at[pl.ds(base, CHUNK)], idx_v.at[b], gsem.at[b])

    def _wait_gather(ch, b):
        base = ch * CHUNK
        pltpu.make_async_copy(
            vals_hbm.at[pl.ds(base, CHUNK)], rows_v.at[b], gsem.at[b]).wait()
        pltpu.make_async_copy(
            idx_hbm.at[pl.ds(base, CHUNK)], idx_v.at[b], gsem.at[b]).wait()

    _start_gather(wid, 0)

    def _pair(kk, _):
        k2 = kk * 2
        for b in (0, 1):
            k = k2 + b
            ch = wid + k * NW
            nxt = ch + NW

            @pl.when(ch < NCH_FULL)
            def _():
                _wait_gather(ch, b)

                @pl.when(nxt < NCH_FULL)
                def _():
                    _start_gather(nxt, 1 - b)

                pltpu.async_copy(ones_v, acc_cnt.at[idx_v.at[b]], csem, add=True)
                pltpu.sync_copy(rows_v.at[b], acc_vals.at[idx_v.at[b]], add=True)
                pltpu.make_async_copy(
                    ones_v, acc_cnt.at[idx_v.at[b]], csem).wait()

        return 0

    lax.fori_loop(0, (KMAX + 1) // 2, _pair, 0)

    @pl.when(wid == NW - 1)
    def _():
        base = NCH_FULL * CHUNK
        pltpu.sync_copy(vals_hbm.at[pl.ds(base, TAIL)],
                        rows_v.at[0].at[pl.ds(0, TAIL)])
        pltpu.sync_copy(idx_hbm.at[pl.ds(base, TAIL)], idx_tail_v)
        pltpu.sync_copy(rows_v.at[0].at[pl.ds(0, TAIL)],
                        acc_vals.at[idx_tail_v], add=True)
        pltpu.sync_copy(ones_v.at[pl.ds(0, TAIL)],
                        acc_cnt.at[idx_tail_v], add=True)

    plsc.subcore_barrier()

    sl = pl.ds(sid * SEG_PER_TILE, SEG_PER_TILE)
    pltpu.sync_copy(acc_vals.at[sl], out_vals.at[cid, sl])
    pltpu.sync_copy(acc_cnt.at[sl], out_cnt.at[cid, sl])


def _sc_pool(vals, idx):
    mesh = plsc.VectorSubcoreMesh(core_axis_name="c", subcore_axis_name="s")
    f = pl.kernel(
        _sc_pool_body,
        out_type=(
            jax.ShapeDtypeStruct((2, NUM_SEG, D_OUT), jnp.float32),
            jax.ShapeDtypeStruct((2, NUM_SEG, D_OUT), jnp.float32),
        ),
        mesh=mesh,
        scratch_types=[
            pltpu.VMEM((2, CHUNK, D_OUT), jnp.float32),
            pltpu.VMEM((2, CHUNK), jnp.int32),
            pltpu.VMEM((TAIL,), jnp.int32),
            pltpu.VMEM((CHUNK, D_OUT), jnp.float32),
            pltpu.VMEM((SEG_PER_TILE, D_OUT), jnp.float32),
            pltpu.VMEM_SHARED((NUM_SEG, D_OUT), jnp.float32),
            pltpu.VMEM_SHARED((NUM_SEG, D_OUT), jnp.float32),
            pltpu.SemaphoreType.DMA((2,)),
            pltpu.SemaphoreType.DMA,
        ],
    )
    return f(vals, idx)


def _merge_body(av_ref, ac_ref, zsum_ref, out_ref):
    v = av_ref[0] + av_ref[1]
    c = ac_ref[0, :, 0:1] + ac_ref[1, :, 0:1]
    zt = zsum_ref[0]
    out_ref[...] = v / (zt * jnp.maximum(c, 1.0))


def _merge(acc_vals, acc_cnt, zsum):
    return pl.pallas_call(
        _merge_body,
        grid=(1,),
        in_specs=[
            pl.BlockSpec((2, NUM_SEG, D_OUT), lambda i: (0, 0, 0)),
            pl.BlockSpec((2, NUM_SEG, D_OUT), lambda i: (0, 0, 0)),
            pl.BlockSpec(memory_space=pltpu.SMEM),
        ],
        out_specs=pl.BlockSpec((NUM_SEG, D_OUT), lambda i: (0, 0)),
        out_shape=jax.ShapeDtypeStruct((NUM_SEG, D_OUT), jnp.float32),
    )(acc_vals, acc_cnt, zsum)


def kernel(z, batch_index, W1, b1, W2, b2):
    seg = batch_index.astype(jnp.int32)
    z16 = z.astype(jnp.bfloat16)
    W116 = W1.astype(jnp.bfloat16)
    b1r = b1.reshape(1, HID)
    m = _scores_max(z16, W116, b1r, W2)
    vals, zsum = _weighted_vals(z16, W116, b1r, W2, b2.reshape(1), m.reshape(1))
    acc_vals, acc_cnt = _sc_pool(vals, seg)
    return _merge(acc_vals, acc_cnt, zsum.reshape(1))
